# baseline (device time: 11467 ns/iter reference)
import jax
import jax.numpy as jnp
from jax import lax
from jax.experimental import pallas as pl
from jax.experimental.pallas import tpu as pltpu

C = 4


def kernel(x):
    m, n = x.shape
    mc = m // C

    def body(x_ref, out_ref, comm1_ref,
             send1_sems, recv1_sems, send2_sems, recv2_sems):
        my_x = lax.axis_index("x")
        my_y = lax.axis_index("y")
        x_partner = (1 - my_x, my_y)
        y_partner = (my_x, 1 - my_y)

        barrier_sem = pltpu.get_barrier_semaphore()
        for nbr in (x_partner, y_partner):
            pl.semaphore_signal(
                barrier_sem, inc=1,
                device_id=nbr, device_id_type=pl.DeviceIdType.MESH,
            )
        pl.semaphore_wait(barrier_sem, 2)

        rows = lambda c: pl.ds(c * mc, mc)

        def run(my_col):
            rdma1 = []
            for c in range(C):
                r = pltpu.make_async_remote_copy(
                    src_ref=x_ref.at[rows(c)],
                    dst_ref=comm1_ref.at[rows(c)],
                    send_sem=send1_sems.at[c],
                    recv_sem=recv1_sems.at[c],
                    device_id=x_partner,
                    device_id_type=pl.DeviceIdType.MESH,
                )
                r.start()
                rdma1.append(r)

            rdma2 = []
            for c in range(C):
                rdma1[c].wait_recv()
                out_ref[rows(c), pl.ds(my_col, n)] = (
                    x_ref[rows(c), :] + comm1_ref[rows(c), :]
                )
                r = pltpu.make_async_remote_copy(
                    src_ref=out_ref.at[rows(c), pl.ds(my_col, n)],
                    dst_ref=out_ref.at[rows(c), pl.ds(my_col, n)],
                    send_sem=send2_sems.at[c],
                    recv_sem=recv2_sems.at[c],
                    device_id=y_partner,
                    device_id_type=pl.DeviceIdType.MESH,
                )
                r.start()
                rdma2.append(r)

            for c in range(C):
                rdma2[c].wait_recv()
            for c in range(C):
                rdma1[c].wait_send()
                rdma2[c].wait_send()

        pl.when(my_y == 0)(lambda: run(0))
        pl.when(my_y == 1)(lambda: run(n))

    return pl.pallas_call(
        body,
        out_shape=jax.ShapeDtypeStruct((m, 2 * n), x.dtype),
        in_specs=[pl.BlockSpec(memory_space=pltpu.VMEM)],
        out_specs=pl.BlockSpec(memory_space=pltpu.VMEM),
        scratch_shapes=[
            pltpu.VMEM((m, n), x.dtype),
            pltpu.SemaphoreType.DMA((C,)),
            pltpu.SemaphoreType.DMA((C,)),
            pltpu.SemaphoreType.DMA((C,)),
            pltpu.SemaphoreType.DMA((C,)),
        ],
        compiler_params=pltpu.CompilerParams(collective_id=0),
    )(x)
